# Initial kernel scaffold; baseline (speedup 1.0000x reference)
import jax
import jax.numpy as jnp
from jax.experimental import pallas as pl


def kernel(img_feat, cloud, W1, b1, W2, b2, Wp1, bp1, Wp2, bp2, Wf, bf):
    raise NotImplementedError("write your pallas kernel here")


# speedup vs baseline: 3.0538x; 3.0538x over previous
import functools

import jax
import jax.numpy as jnp
from jax import lax
from jax.experimental import pallas as pl
from jax.experimental.pallas import tpu as pltpu
from jax.experimental.pallas import tpu_sc as plsc

_N = 4096
_K = 16
_QB = 16
_NC = 2
_NS = 16
_NW = _NC * _NS
_QPW = _N // _NW
_QC = 8


def _stage1(ptsq_ref, ptst_ref, featq_ref, w1_ref, w2_ref, b1_ref, b2_ref,
            inds_ref, g_ref):
    pq = ptsq_ref[...]
    pt = ptst_ref[...]
    sqq = jnp.sum(pq * pq, axis=1, keepdims=True)
    sqa = jnp.sum(pt * pt, axis=0, keepdims=True)
    dot = lax.dot_general(pq, pt, (((1,), (0,)), ((), ())),
                          preferred_element_type=jnp.float32)
    d2 = sqq + sqa - 2.0 * dot
    iota = lax.broadcasted_iota(jnp.int32, (_QB, _N), 1)
    cur = d2
    icols = []
    for _ in range(_K):
        m = jnp.min(cur, axis=1, keepdims=True)
        idx = jnp.min(jnp.where(cur == m, iota, jnp.int32(_N)),
                      axis=1, keepdims=True)
        icols.append(idx)
        cur = jnp.where(iota == idx, jnp.float32(jnp.inf), cur)
    inds_ref[...] = jnp.concatenate(icols, axis=1)
    h = lax.dot_general(featq_ref[...], w1_ref[...], (((1,), (1,)), ((), ())),
                        preferred_element_type=jnp.float32) + b1_ref[...]
    g = lax.dot_general(h, w2_ref[...], (((1,), (1,)), ((), ())),
                        preferred_element_type=jnp.float32) + b2_ref[...]
    g_ref[...] = g


def _run_stage1(pts, ptsT, feat, W1, W2, b1r, b2r, interpret=False):
    grid = (_N // _QB,)
    return pl.pallas_call(
        _stage1,
        grid=grid,
        in_specs=[
            pl.BlockSpec((_QB, 3), lambda i: (i, 0)),
            pl.BlockSpec((3, _N), lambda i: (0, 0)),
            pl.BlockSpec((_QB, 32), lambda i: (i, 0)),
            pl.BlockSpec((64, 32), lambda i: (0, 0)),
            pl.BlockSpec((128, 64), lambda i: (0, 0)),
            pl.BlockSpec((1, 64), lambda i: (0, 0)),
            pl.BlockSpec((1, 128), lambda i: (0, 0)),
        ],
        out_specs=[
            pl.BlockSpec((_QB, _K), lambda i: (i, 0)),
            pl.BlockSpec((_QB, 128), lambda i: (i, 0)),
        ],
        out_shape=[
            jax.ShapeDtypeStruct((_N, _K), jnp.int32),
            jax.ShapeDtypeStruct((_N, 128), jnp.float32),
        ],
        interpret=interpret,
    )(pts, ptsT, feat, W1, W2, b1r, b2r)


def _sc_body(inds_hbm, pp_hbm, xt_hbm, yt_hbm, zt_hbm, g_hbm, out_hbm,
             idx_v, rows_v, xs_v, ys_v, zs_v, qp_v, acc_v,
             sem, semx, semy, semz):
    wid = lax.axis_index("s") * _NC + lax.axis_index("c")
    qbase = wid * _QPW
    magic = jnp.full((16,), 0x1fbd1df5, jnp.int32)

    def chunk_body(ci, carry):
        q0 = qbase + ci * _QC
        pltpu.sync_copy(inds_hbm.at[pl.ds(q0 * _K, _QC * _K)], idx_v)
        cp = pltpu.async_copy(g_hbm.at[idx_v], rows_v, sem)
        cpx = pltpu.async_copy(xt_hbm.at[idx_v], xs_v, semx)
        cpy = pltpu.async_copy(yt_hbm.at[idx_v], ys_v, semy)
        cpz = pltpu.async_copy(zt_hbm.at[idx_v], zs_v, semz)
        pltpu.sync_copy(pp_hbm.at[pl.ds(q0, _QC), :], qp_v)
        cp.wait()
        cpx.wait()
        cpy.wait()
        cpz.wait()
        for q in range(_QC):
            qrow = qp_v[q, :]
            xs = xs_v[pl.ds(q * _K, 16)]
            ys = ys_v[pl.ds(q * _K, 16)]
            zs = zs_v[pl.ds(q * _K, 16)]
            dx = xs - jnp.broadcast_to(qrow[0], (16,))
            dy = ys - jnp.broadcast_to(qrow[1], (16,))
            dz = zs - jnp.broadcast_to(qrow[2], (16,))
            d2e = dx * dx + dy * dy + dz * dz + jnp.float32(1e-12)
            y0 = lax.bitcast_convert_type(
                magic + (lax.bitcast_convert_type(d2e, jnp.int32) >> 1),
                jnp.float32)
            y1 = 0.5 * (y0 + d2e / y0)
            y2 = 0.5 * (y1 + d2e / y1)
            dist = 0.5 * (y2 + d2e / y2)
            e = jnp.exp(-dist)
            s = e[0]
            for k in range(1, _K):
                s = s + e[k]
            rinv = jnp.full((16,), 1.0, jnp.float32) / jnp.broadcast_to(s, (16,))
            acc = [None] * 8
            for k in range(_K):
                sv = jnp.broadcast_to(e[k], (16,))
                r = q * _K + k
                for f in range(8):
                    term = rows_v[r, pl.ds(f * 16, 16)] * sv
                    acc[f] = term if k == 0 else jnp.maximum(acc[f], term)
            for f in range(8):
                acc_v[q, pl.ds(f * 16, 16)] = acc[f] * rinv
        pltpu.sync_copy(acc_v, out_hbm.at[pl.ds(q0, _QC), :])
        return carry

    lax.fori_loop(0, _QPW // _QC, chunk_body, 0)


def _run_stage2(inds_flat, pts_pad, xt, yt, zt, g):
    mesh = plsc.VectorSubcoreMesh(core_axis_name="c", subcore_axis_name="s")
    fn = functools.partial(
        pl.kernel,
        mesh=mesh,
        out_type=jax.ShapeDtypeStruct((_N, 128), jnp.float32),
        scratch_types=[
            pltpu.VMEM((_QC * _K,), jnp.int32),
            pltpu.VMEM((_QC * _K, 128), jnp.float32),
            pltpu.VMEM((_QC * _K,), jnp.float32),
            pltpu.VMEM((_QC * _K,), jnp.float32),
            pltpu.VMEM((_QC * _K,), jnp.float32),
            pltpu.VMEM((_QC, 16), jnp.float32),
            pltpu.VMEM((_QC, 128), jnp.float32),
            pltpu.SemaphoreType.DMA,
            pltpu.SemaphoreType.DMA,
            pltpu.SemaphoreType.DMA,
            pltpu.SemaphoreType.DMA,
        ],
    )(_sc_body)
    return fn(inds_flat, pts_pad, xt, yt, zt, g)


def _stage3(s_ref, ptst_ref, wf_ref, wp1_ref, wp2_ref, bp1_ref, bp2_ref,
            bf_ref, out_ref):
    wf = wf_ref[...]
    wfa = wf[:, :128]
    wfb = wf[:, 128:]
    hp = lax.dot_general(wp1_ref[...], ptst_ref[...], (((1,), (0,)), ((), ())),
                         preferred_element_type=jnp.float32) + bp1_ref[...]
    pfull = lax.dot_general(wp2_ref[...], hp, (((1,), (0,)), ((), ())),
                            preferred_element_type=jnp.float32) + bp2_ref[...]
    outa = lax.dot_general(wfa, s_ref[...], (((1,), (1,)), ((), ())),
                           preferred_element_type=jnp.float32)
    outb = lax.dot_general(wfb, pfull, (((1,), (0,)), ((), ())),
                           preferred_element_type=jnp.float32)
    out_ref[...] = outa + outb + bf_ref[...]


def _run_stage3(surround, ptsT, Wf, Wp1, Wp2, bp1c, bp2c, bfc, interpret=False):
    return pl.pallas_call(
        _stage3,
        out_shape=jax.ShapeDtypeStruct((128, _N), jnp.float32),
        interpret=interpret,
    )(surround, ptsT, Wf, Wp1, Wp2, bp1c, bp2c, bfc)


def kernel(img_feat, cloud, W1, b1, W2, b2, Wp1, bp1, Wp2, bp2, Wf, bf):
    pts = cloud[0]
    feat = jnp.transpose(img_feat[0])
    ptsT = jnp.transpose(pts)
    inds, g = _run_stage1(pts, ptsT, feat, W1, W2,
                          b1[None, :], b2[None, :])
    pts_pad = jnp.concatenate([pts, jnp.zeros((_N, 13), jnp.float32)], axis=1)
    surround = _run_stage2(inds.reshape(-1), pts_pad,
                           pts[:, 0], pts[:, 1], pts[:, 2], g)
    outT = _run_stage3(surround, ptsT, Wf, Wp1, Wp2,
                       bp1[:, None], bp2[:, None], bf[:, None])
    return outT[None]


# speedup vs baseline: 13.4597x; 4.4075x over previous
import functools

import jax
import jax.numpy as jnp
from jax import lax
from jax.experimental import pallas as pl
from jax.experimental.pallas import tpu as pltpu
from jax.experimental.pallas import tpu_sc as plsc

_N = 4096
_K = 16
_QB = 128
_NC = 2
_NS = 16
_NW = _NC * _NS
_QPW = _N // _NW
_QC = 8


def _tables(pts_ref, ptst_ref, feat_ref, w1_ref, w2_ref, b1_ref, b2_ref,
            g_ref, ppad_ref, xt_ref, yt_ref, zt_ref):
    pq = pts_ref[...]
    pt = ptst_ref[...]
    h = lax.dot_general(feat_ref[...], w1_ref[...], (((0,), (1,)), ((), ())),
                        preferred_element_type=jnp.float32) + b1_ref[...]
    g = lax.dot_general(h, w2_ref[...], (((1,), (1,)), ((), ())),
                        preferred_element_type=jnp.float32) + b2_ref[...]
    g_ref[...] = g
    ppad_ref[...] = jnp.concatenate(
        [pq, jnp.zeros((_N, 13), jnp.float32)], axis=1)
    xt_ref[...] = pt[0:1, :]
    yt_ref[...] = pt[1:2, :]
    zt_ref[...] = pt[2:3, :]


def _run_tables(pts, ptsT, feat, W1, W2, b1r, b2r):
    return pl.pallas_call(
        _tables,
        out_shape=[
            jax.ShapeDtypeStruct((_N, 128), jnp.float32),
            jax.ShapeDtypeStruct((_N, 16), jnp.float32),
            jax.ShapeDtypeStruct((1, _N), jnp.float32),
            jax.ShapeDtypeStruct((1, _N), jnp.float32),
            jax.ShapeDtypeStruct((1, _N), jnp.float32),
        ],
    )(pts, ptsT, feat, W1, W2, b1r, b2r)


def _stage1(ptsq_ref, ptst_ref, inds_ref):
    pq = ptsq_ref[...]
    pt = ptst_ref[...]
    sqq = jnp.sum(pq * pq, axis=1, keepdims=True)
    sqa = jnp.sum(pt * pt, axis=0, keepdims=True)
    dot = lax.dot_general(pq, pt, (((1,), (0,)), ((), ())),
                          preferred_element_type=jnp.float32)
    d2 = sqq + sqa - 2.0 * dot
    iota = lax.broadcasted_iota(jnp.int32, (_QB, _N), 1).astype(jnp.float32)
    cur = d2
    icols = []
    for _ in range(_K):
        m = jnp.min(cur, axis=1, keepdims=True)
        idx = jnp.min(jnp.where(cur == m, iota, jnp.float32(_N)),
                      axis=1, keepdims=True)
        icols.append(idx)
        cur = jnp.where(iota == idx, jnp.float32(jnp.inf), cur)
    inds_ref[...] = jnp.concatenate(icols, axis=1).astype(jnp.int32)


def _run_stage1(pts, ptsT, r0, nrows, interpret=False):
    grid = (nrows // _QB,)
    ob = r0 // _QB
    return pl.pallas_call(
        _stage1,
        grid=grid,
        in_specs=[
            pl.BlockSpec((_QB, 3), lambda i: (i + ob, 0)),
            pl.BlockSpec((3, _N), lambda i: (0, 0)),
        ],
        out_specs=[
            pl.BlockSpec((_QB, _K), lambda i: (i, 0)),
        ],
        out_shape=[
            jax.ShapeDtypeStruct((nrows, _K), jnp.int32),
        ],
        interpret=interpret,
    )(pts, ptsT)[0]


_NCH = _QPW // _QC
_CI = _QC * _K


def _make_sc_body(r0, qpw):
    nch = qpw // _QC

    def _sc_body(inds_hbm, pp_hbm, xt_hbm, yt_hbm, zt_hbm, g_hbm, out_hbm,
                 idxall_v, qpall_v, rows_v, xs_v, ys_v, zs_v, acc_v,
                 semg, semx, semy, semz):
        wid = lax.axis_index("s") * _NC + lax.axis_index("c")
        obase = wid * qpw
        qbase = r0 + obase
        magic = jnp.full((16,), 0x1fbd1df5, jnp.int32)

        pltpu.sync_copy(inds_hbm.at[pl.ds(obase * _K, qpw * _K)], idxall_v)
        pltpu.sync_copy(pp_hbm.at[pl.ds(qbase, qpw), :], qpall_v)

        def issue(c):
            p = c & 1
            idxs = idxall_v.at[pl.ds(c * _CI, _CI)]
            pltpu.async_copy(g_hbm.at[idxs], rows_v.at[p], semg.at[p])
            pltpu.async_copy(xt_hbm.at[idxs], xs_v.at[p], semx.at[p])
            pltpu.async_copy(yt_hbm.at[idxs], ys_v.at[p], semy.at[p])
            pltpu.async_copy(zt_hbm.at[idxs], zs_v.at[p], semz.at[p])

        def wait(c):
            p = c & 1
            idxs = idxall_v.at[pl.ds(c * _CI, _CI)]
            pltpu.make_async_copy(g_hbm.at[idxs], rows_v.at[p],
                                  semg.at[p]).wait()
            pltpu.make_async_copy(xt_hbm.at[idxs], xs_v.at[p],
                                  semx.at[p]).wait()
            pltpu.make_async_copy(yt_hbm.at[idxs], ys_v.at[p],
                                  semy.at[p]).wait()
            pltpu.make_async_copy(zt_hbm.at[idxs], zs_v.at[p],
                                  semz.at[p]).wait()

        def compute(c, p):
            def q_body(q, carry):
                qi = c * _QC + q
                qrow = qpall_v[qi, :]
                xs = xs_v[p, pl.ds(q * _K, 16)]
                ys = ys_v[p, pl.ds(q * _K, 16)]
                zs = zs_v[p, pl.ds(q * _K, 16)]
                dx = xs - jnp.broadcast_to(qrow[0], (16,))
                dy = ys - jnp.broadcast_to(qrow[1], (16,))
                dz = zs - jnp.broadcast_to(qrow[2], (16,))
                d2e = dx * dx + dy * dy + dz * dz + jnp.float32(1e-12)
                y0 = lax.bitcast_convert_type(
                    magic + (lax.bitcast_convert_type(d2e, jnp.int32) >> 1),
                    jnp.float32)
                y1 = 0.5 * (y0 + d2e / y0)
                y2 = 0.5 * (y1 + d2e / y1)
                dist = 0.5 * (y2 + d2e / y2)
                e = jnp.exp(-dist)
                s = e[0]
                for k in range(1, _K):
                    s = s + e[k]
                rinv = (jnp.full((16,), 1.0, jnp.float32)
                        / jnp.broadcast_to(s, (16,)))
                acc = [None] * 8
                for k in range(_K):
                    sv = jnp.broadcast_to(e[k], (16,))
                    r = q * _K + k
                    for f in range(8):
                        term = rows_v[p, r, pl.ds(f * 16, 16)] * sv
                        acc[f] = term if k == 0 else jnp.maximum(acc[f], term)
                for f in range(8):
                    acc_v[qi, pl.ds(f * 16, 16)] = acc[f] * rinv
                return carry

            lax.fori_loop(0, _QC, q_body, 0)

        issue(0)

        def chunk_body(c, carry):
            issue(c + 1)
            wait(c)
            compute(c, c & 1)
            return carry

        lax.fori_loop(0, nch - 1, chunk_body, 0)
        wait(nch - 1)
        compute(nch - 1, (nch - 1) & 1)
        pltpu.sync_copy(acc_v, out_hbm.at[pl.ds(obase, qpw), :])

    return _sc_body


def _run_stage2(inds_flat, pts_pad, xt, yt, zt, g, r0, nrows):
    qpw = nrows // _NW
    mesh = plsc.VectorSubcoreMesh(core_axis_name="c", subcore_axis_name="s")
    fn = functools.partial(
        pl.kernel,
        mesh=mesh,
        out_type=jax.ShapeDtypeStruct((nrows, 128), jnp.float32),
        scratch_types=[
            pltpu.VMEM((qpw * _K,), jnp.int32),
            pltpu.VMEM((qpw, 16), jnp.float32),
            pltpu.VMEM((2, _CI, 128), jnp.float32),
            pltpu.VMEM((2, _CI), jnp.float32),
            pltpu.VMEM((2, _CI), jnp.float32),
            pltpu.VMEM((2, _CI), jnp.float32),
            pltpu.VMEM((qpw, 128), jnp.float32),
            pltpu.SemaphoreType.DMA((2,)),
            pltpu.SemaphoreType.DMA((2,)),
            pltpu.SemaphoreType.DMA((2,)),
            pltpu.SemaphoreType.DMA((2,)),
        ],
    )(_make_sc_body(r0, qpw))
    return fn(inds_flat, pts_pad, xt, yt, zt, g)


def _stage3(s0_ref, s1_ref, ptst_ref, wf_ref, wp1_ref, wp2_ref, bp1_ref,
            bp2_ref, bf_ref, out_ref):
    wf = wf_ref[...]
    wfa = wf[:, :128]
    wfb = wf[:, 128:]
    hp = lax.dot_general(wp1_ref[...], ptst_ref[...], (((1,), (0,)), ((), ())),
                         preferred_element_type=jnp.float32) + bp1_ref[...]
    pfull = lax.dot_general(wp2_ref[...], hp, (((1,), (0,)), ((), ())),
                            preferred_element_type=jnp.float32) + bp2_ref[...]
    outa0 = lax.dot_general(wfa, s0_ref[...], (((1,), (1,)), ((), ())),
                            preferred_element_type=jnp.float32)
    outa1 = lax.dot_general(wfa, s1_ref[...], (((1,), (1,)), ((), ())),
                            preferred_element_type=jnp.float32)
    outb = lax.dot_general(wfb, pfull, (((1,), (0,)), ((), ())),
                           preferred_element_type=jnp.float32)
    rest = outb + bf_ref[...]
    out_ref[:, : _N // 2] = outa0 + rest[:, : _N // 2]
    out_ref[:, _N // 2:] = outa1 + rest[:, _N // 2:]


def _run_stage3(s0, s1, ptsT, Wf, Wp1, Wp2, bp1c, bp2c, bfc, interpret=False):
    return pl.pallas_call(
        _stage3,
        out_shape=jax.ShapeDtypeStruct((128, _N), jnp.float32),
        interpret=interpret,
    )(s0, s1, ptsT, Wf, Wp1, Wp2, bp1c, bp2c, bfc)


def kernel(img_feat, cloud, W1, b1, W2, b2, Wp1, bp1, Wp2, bp2, Wf, bf):
    pts = cloud[0]
    featT = img_feat[0]
    ptsT = jnp.transpose(pts)
    g, pts_pad, xt, yt, zt = _run_tables(pts, ptsT, featT, W1, W2,
                                         b1[None, :], b2[None, :])
    nsplit = 2
    part = _N // nsplit
    parts = []
    for si in range(nsplit):
        inds_s = _run_stage1(pts, ptsT, si * part, part)
        parts.append(_run_stage2(inds_s.reshape(-1), pts_pad,
                                 xt[0], yt[0], zt[0], g, si * part, part))
    outT = _run_stage3(parts[0], parts[1], ptsT, Wf, Wp1, Wp2,
                       bp1[:, None], bp2[:, None], bf[:, None])
    return outT[None]
